# baseline (speedup 1.0000x reference)
import functools

import jax
import jax.numpy as jnp
from jax import lax
from jax.experimental import pallas as pl
from jax.experimental.pallas import tpu as pltpu
from jax.experimental.pallas import tpu_sc as plsc

_NC = 2
_NS = 16
_NW = _NC * _NS


def _edge_aggregate(x, src, dst, edge_attr, coef=None):
    N, D = x.shape
    affine = coef is not None
    E = src.shape[0]
    B = 64
    EPW = E // _NW
    FULL = EPW // B
    TAIL = EPW - FULL * B
    Q = 4
    SR = 2
    SU = FULL // Q
    PER = 24
    assert EPW * _NW == E and TAIL % 8 == 0
    assert FULL % Q == 0 and FULL >= 3 * Q
    NI = ((FULL - 2 * Q - 1) // PER) * PER
    while NI > FULL - 2 * Q - 3:
        NI -= PER
    assert NI >= 0
    NP = ((N + _NS * 8 - 1) // (_NS * 8)) * (_NS * 8)
    RPT = NP // _NS
    assert RPT % 8 == 0 and (RPT % B) % 8 == 0
    L = 16
    R = 3
    R2 = 2

    mesh = plsc.VectorSubcoreMesh(core_axis_name="c", subcore_axis_name="s")

    scratch = [pltpu.VMEM_SHARED((NP, D), jnp.float32)]
    scratch += [pltpu.VMEM((Q * B,), jnp.int32) for _ in range(SR)]
    scratch += [pltpu.VMEM((Q * B,), jnp.int32) for _ in range(SR)]
    scratch += [pltpu.VMEM((B,), jnp.int32) for _ in range(R)]
    scratch += [pltpu.VMEM((B, D), jnp.float32) for _ in range(R2)]
    scratch += [pltpu.VMEM((B, D), jnp.float32) for _ in range(R)]
    if TAIL:
        scratch += [pltpu.VMEM((TAIL,), jnp.int32),
                    pltpu.VMEM((TAIL,), jnp.int32)]
    if affine:
        scratch += [pltpu.VMEM((D,), jnp.float32),
                    pltpu.VMEM((D,), jnp.float32)]
    scratch += [pltpu.SemaphoreType.DMA] * (SR + 2 * R + R2 + 1)

    @functools.partial(
        pl.kernel,
        out_type=jax.ShapeDtypeStruct((_NC, NP, D), jnp.float32),
        mesh=mesh,
        scratch_types=scratch,
    )
    def k(x_hbm, src_hbm, dst_hbm, ea_hbm, *rest):
        if affine:
            sc_hbm, of_hbm = rest[0], rest[1]
            rest = rest[2:]
        out_hbm, aggr_sh = rest[0], rest[1]
        sc = rest[2:]
        srcq, sc = sc[:SR], sc[SR:]
        dstq, sc = sc[:SR], sc[SR:]
        dstS, sc = sc[:R], sc[R:]
        eab, sc = sc[:R2], sc[R2:]
        xrb, sc = sc[:R], sc[R:]
        if TAIL:
            (srct, dstt), sc = sc[:2], sc[2:]
        if affine:
            (scale_v, off_v), sc = sc[:2], sc[2:]
            pltpu.sync_copy(sc_hbm, scale_v)
            pltpu.sync_copy(of_hbm, off_v)
        isem, sc = sc[:SR], sc[SR:]
        gsem, sc = sc[:R], sc[R:]
        easem, sc = sc[:R2], sc[R2:]
        ssem = sc[:R]
        tsem = sc[R]

        c = lax.axis_index("c")
        s = lax.axis_index("s")
        wid = c * _NS + s
        base = wid * EPW

        @plsc.parallel_loop(0, B)
        def _(i):
            for j in range(D // L):
                xrb[0][i, pl.ds(j * L, L)] = jnp.zeros((L,), jnp.float32)

        NZ = RPT // B
        REM = RPT - NZ * B
        for t in range(NZ):
            pltpu.async_copy(xrb[0], aggr_sh.at[pl.ds(s * RPT + t * B, B)],
                             tsem)
        if REM:
            pltpu.async_copy(xrb[0].at[pl.ds(0, REM)],
                             aggr_sh.at[pl.ds(s * RPT + NZ * B, REM)], tsem)
        for t in range(NZ):
            pltpu.make_async_copy(xrb[0], aggr_sh.at[pl.ds(0, B)],
                                  tsem).wait()
        if REM:
            pltpu.make_async_copy(xrb[0].at[pl.ds(0, REM)],
                                  aggr_sh.at[pl.ds(0, REM)], tsem).wait()
        plsc.subcore_barrier()

        def fetch(u, su):
            off = base + u * (Q * B)
            pltpu.async_copy(src_hbm.at[pl.ds(off, Q * B)], srcq[su],
                             isem[su])
            pltpu.async_copy(dst_hbm.at[pl.ds(off, Q * B)], dstq[su],
                             isem[su])

        def wait_fetch(su):
            pltpu.make_async_copy(src_hbm.at[pl.ds(0, Q * B)], srcq[su],
                                  isem[su]).wait()
            pltpu.make_async_copy(dst_hbm.at[pl.ds(0, Q * B)], dstq[su],
                                  isem[su]).wait()

        def issue_ge(g, b, be, q, su):
            off = base + g * B
            pltpu.async_copy(ea_hbm.at[pl.ds(off, B)], eab[be], easem[be])
            pltpu.async_copy(x_hbm.at[srcq[su].at[pl.ds(q * B, B)]], xrb[b],
                             gsem[b])

        def wait_ge(b, be, q, su):
            pltpu.make_async_copy(ea_hbm.at[pl.ds(0, B)], eab[be],
                                  easem[be]).wait()
            pltpu.make_async_copy(x_hbm.at[srcq[su].at[pl.ds(q * B, B)]],
                                  xrb[b], gsem[b]).wait()

        def issue_scatter(b):
            pltpu.make_async_copy(xrb[b], aggr_sh.at[dstS[b]],
                                  ssem[b]).start(add=True)

        def wait_scatter(b):
            pltpu.make_async_copy(xrb[b], aggr_sh.at[dstS[b]],
                                  ssem[b]).wait()

        def save_dst(b, q, su):
            for j in range(B // L):
                dstS[b][pl.ds(j * L, L)] = dstq[su][pl.ds(q * B + j * L, L)]

        def compute(xr, ea, n):
            if affine:
                sjs = [scale_v[pl.ds(j * L, L)] for j in range(D // L)]
                ojs = [off_v[pl.ds(j * L, L)] for j in range(D // L)]

                @plsc.parallel_loop(0, n, unroll=1)
                def _(i):
                    for j in range(D // L):
                        sl = pl.ds(j * L, L)
                        y = jnp.maximum(xr[i, sl] * sjs[j] + ojs[j], 0.0)
                        xr[i, sl] = jnp.maximum(y + ea[i, sl], 0.0)
            else:
                @plsc.parallel_loop(0, n, unroll=2)
                def _(i):
                    for j in range(D // L):
                        sl = pl.ds(j * L, L)
                        xr[i, sl] = jnp.maximum(xr[i, sl] + ea[i, sl], 0.0)

        def chunk_body(g, gi, first, last):
            b, be, q, su = gi % R, gi % R2, gi % Q, (gi // Q) % SR
            b1 = (b + 1) % R
            be1 = (be + 1) % R2
            su1 = ((gi + 1) // Q) % SR
            if not last:
                if not first:
                    wait_scatter(b1)
                if q == Q - 1:
                    if not isinstance(g, int) or (g + 1) // Q <= SU - 1:
                        wait_fetch(su1)
                issue_ge(g + 1, b1, be1, (q + 1) % Q, su1)
                if q == 1:
                    if not isinstance(g, int) or 2 <= (g + 3) // Q <= SU - 1:
                        fetch((g + 3) // Q, ((gi + 3) // Q) % SR)
            wait_ge(b, be, q, su)
            save_dst(b, q, su)
            compute(xrb[b], eab[be], B)
            issue_scatter(b)

        fetch(0, 0)
        fetch(1, 1)
        wait_fetch(0)
        issue_ge(0, 0, 0, 0, 0)
        for g in range(2 * Q):
            chunk_body(g, g, g < 2, False)

        def group(p, carry):
            g0 = 2 * Q + p * PER
            for t in range(PER):
                chunk_body(g0 + t, 2 * Q + t, False, False)
            return carry
        lax.fori_loop(0, NI // PER, group, 0)

        for g in range(2 * Q + NI, FULL):
            chunk_body(g, g, False, g == FULL - 1)

        if TAIL:
            wait_scatter(0)
            off = base + FULL * B
            pltpu.sync_copy(src_hbm.at[pl.ds(off, TAIL)], srct)
            pltpu.sync_copy(dst_hbm.at[pl.ds(off, TAIL)], dstt)
            pltpu.sync_copy(ea_hbm.at[pl.ds(off, TAIL)],
                            eab[0].at[pl.ds(0, TAIL)])
            pltpu.async_copy(x_hbm.at[srct], xrb[0].at[pl.ds(0, TAIL)],
                             tsem).wait()
            compute(xrb[0], eab[0], TAIL)
            pltpu.sync_copy(xrb[0].at[pl.ds(0, TAIL)], aggr_sh.at[dstt],
                            add=True)

        for b in range(R):
            if TAIL and b == 0:
                continue
            wait_scatter(b)
        plsc.subcore_barrier()
        pltpu.sync_copy(aggr_sh.at[pl.ds(s * RPT, RPT)],
                        out_hbm.at[c, pl.ds(s * RPT, RPT)])

    if affine:
        return k(x, src, dst, edge_attr, coef[0], coef[1])
    return k(x, src, dst, edge_attr)


_BR = 400


def _mlp_stats_body(count, nsteps, in_affine, out_coef, eps_ref, x_ref, *refs):
    refs = list(refs)
    coef_in = refs.pop(0) if in_affine else None
    a0_ref, a1_ref, wa_ref, ba_ref, wb_ref, bb_ref = refs[:6]
    refs = refs[6:]
    if out_coef:
        lnw_ref, lnb_ref = refs[:2]
        refs = refs[2:]
    h_ref, s_ref, ss_ref = refs[:3]
    coef_out = refs[3] if out_coef else None

    xin = x_ref[...]
    if in_affine:
        xin = jnp.maximum(xin * coef_in[0:1, :] + coef_in[1:2, :], 0.0)
    t = (1.0 + eps_ref[0]) * xin + a0_ref[...] + a1_ref[...]
    u = jnp.dot(t, wa_ref[...], preferred_element_type=jnp.float32)
    u = jnp.maximum(u + ba_ref[...], 0.0)
    h = jnp.dot(u, wb_ref[...], preferred_element_type=jnp.float32)
    h = h + bb_ref[...]
    h_ref[...] = h
    hp = h.reshape(h.shape[0] // 8, 8, h.shape[1])

    @pl.when(pl.program_id(0) == 0)
    def _():
        s_ref[...] = jnp.zeros_like(s_ref)
        ss_ref[...] = jnp.zeros_like(ss_ref)

    s_ref[...] += jnp.sum(hp, axis=0)
    ss_ref[...] += jnp.sum(hp * hp, axis=0)

    if out_coef:
        @pl.when(pl.program_id(0) == nsteps - 1)
        def _():
            mean = jnp.sum(s_ref[...]) / count
            ex2 = jnp.sum(ss_ref[...]) / count
            inv = lax.rsqrt(ex2 - mean * mean + 1e-5)
            scale = inv * lnw_ref[...]
            off = lnb_ref[...] - mean * scale
            pad = jnp.zeros((6, scale.shape[1]), jnp.float32)
            coef_out[...] = jnp.concatenate([scale, off, pad], axis=0)


def _mlp_stats(x, a0, a1, Wa, ba, Wb, bb, eps, ln_w=None, ln_b=None,
               coef=None):
    N, D = x.shape
    G = N // _BR
    out_coef = ln_w is not None
    in_affine = coef is not None
    row_spec = pl.BlockSpec((_BR, D), lambda i: (i, 0))
    full_spec = pl.BlockSpec((D, D), lambda i: (0, 0))
    vec_spec = pl.BlockSpec((1, D), lambda i: (0, 0))
    acc_spec = pl.BlockSpec((8, D), lambda i: (0, 0))

    args = [eps.reshape(1), x]
    in_specs = [pl.BlockSpec(memory_space=pltpu.SMEM), row_spec]
    if in_affine:
        args.append(coef)
        in_specs.append(acc_spec)
    args += [a0, a1, Wa, ba.reshape(1, D), Wb, bb.reshape(1, D)]
    in_specs += [row_spec, row_spec, full_spec, vec_spec, full_spec, vec_spec]
    if out_coef:
        args += [ln_w.reshape(1, D), ln_b.reshape(1, D)]
        in_specs += [vec_spec, vec_spec]

    out_specs = [row_spec, acc_spec, acc_spec]
    out_shape = [
        jax.ShapeDtypeStruct((N, D), jnp.float32),
        jax.ShapeDtypeStruct((8, D), jnp.float32),
        jax.ShapeDtypeStruct((8, D), jnp.float32),
    ]
    if out_coef:
        out_specs.append(acc_spec)
        out_shape.append(jax.ShapeDtypeStruct((8, D), jnp.float32))

    return pl.pallas_call(
        functools.partial(_mlp_stats_body, float(N * D), G, in_affine,
                          out_coef),
        grid=(G,),
        in_specs=in_specs,
        out_specs=out_specs,
        out_shape=out_shape,
    )(*args)


def _norm_body(count, resid, h_ref, s_ref, ss_ref, w_ref, b_ref, *rest):
    if resid:
        x_ref, y_ref = rest
    else:
        (y_ref,) = rest
    mean = jnp.sum(s_ref[...]) / count
    ex2 = jnp.sum(ss_ref[...]) / count
    inv = lax.rsqrt(ex2 - mean * mean + 1e-5)
    y = (h_ref[...] - mean) * inv * w_ref[...] + b_ref[...]
    if resid:
        y = (y + x_ref[...]) * 0.5
    y_ref[...] = jnp.maximum(y, 0.0)


def _norm_relu(h, s, ss, w, b, x=None):
    N, D = h.shape
    G = N // _BR
    resid = x is not None
    row_spec = pl.BlockSpec((_BR, D), lambda i: (i, 0))
    acc_spec = pl.BlockSpec((8, D), lambda i: (0, 0))
    vec_spec = pl.BlockSpec((1, D), lambda i: (0, 0))
    args = [h, s, ss, w.reshape(1, D), b.reshape(1, D)]
    in_specs = [row_spec, acc_spec, acc_spec, vec_spec, vec_spec]
    if resid:
        args.append(x)
        in_specs.append(row_spec)
    return pl.pallas_call(
        functools.partial(_norm_body, float(N * D), resid),
        grid=(G,),
        in_specs=in_specs,
        out_specs=row_spec,
        out_shape=jax.ShapeDtypeStruct((N, D), jnp.float32),
    )(*args)


def kernel(x, edge_index, edge_attr, W1a, b1a, W1b, b1b, eps1, ln1_w, ln1_b,
           W2a, b2a, W2b, b2b, eps2, ln2_w, ln2_b):
    src = edge_index[0]
    dst = edge_index[1]

    agg = _edge_aggregate(x, src, dst, edge_attr)
    h1, s1, ss1, coef1 = _mlp_stats(x, agg[0], agg[1], W1a, b1a, W1b, b1b,
                                    eps1, ln1_w, ln1_b)

    agg2 = _edge_aggregate(h1, src, dst, edge_attr, (coef1[0], coef1[1]))
    h2, s2, ss2 = _mlp_stats(h1, agg2[0], agg2[1], W2a, b2a, W2b, b2b, eps2,
                             coef=coef1)
    out = _norm_relu(h2, s2, ss2, ln2_w, ln2_b, x)
    return out


# speedup vs baseline: 1.0275x; 1.0275x over previous
import functools

import jax
import jax.numpy as jnp
from jax import lax
from jax.experimental import pallas as pl
from jax.experimental.pallas import tpu as pltpu
from jax.experimental.pallas import tpu_sc as plsc

_NC = 2
_NS = 16
_NW = _NC * _NS


def _edge_aggregate(x, src, dst, edge_attr, coef=None):
    N, D = x.shape
    affine = coef is not None
    E = src.shape[0]
    B = 64
    EPW = E // _NW
    FULL = EPW // B
    TAIL = EPW - FULL * B
    assert EPW * _NW == E and TAIL % 8 == 0 and FULL >= 9
    NI = ((FULL - 2) // 6) * 6
    PEEL = FULL - 2 - NI
    if PEEL == 0:
        NI -= 6
        PEEL = 6
    NP = ((N + _NS * 8 - 1) // (_NS * 8)) * (_NS * 8)
    RPT = NP // _NS
    assert RPT % 8 == 0 and (RPT % B) % 8 == 0
    L = 16
    R = 3
    R2 = 2

    mesh = plsc.VectorSubcoreMesh(core_axis_name="c", subcore_axis_name="s")

    scratch = [pltpu.VMEM_SHARED((NP, D), jnp.float32)]
    scratch += [pltpu.VMEM((B,), jnp.int32) for _ in range(R)]
    scratch += [pltpu.VMEM((B,), jnp.int32) for _ in range(R)]
    scratch += [pltpu.VMEM((B,), jnp.int32) for _ in range(R)]
    scratch += [pltpu.VMEM((B, D), jnp.float32) for _ in range(R2)]
    scratch += [pltpu.VMEM((B, D), jnp.float32) for _ in range(R)]
    if TAIL:
        scratch += [pltpu.VMEM((TAIL,), jnp.int32),
                    pltpu.VMEM((TAIL,), jnp.int32)]
    if affine:
        scratch += [pltpu.VMEM((D,), jnp.float32),
                    pltpu.VMEM((D,), jnp.float32)]
    scratch += [pltpu.SemaphoreType.DMA] * (3 * R + R2 + 1)

    @functools.partial(
        pl.kernel,
        out_type=jax.ShapeDtypeStruct((_NC, NP, D), jnp.float32),
        mesh=mesh,
        scratch_types=scratch,
    )
    def k(x_hbm, src_hbm, dst_hbm, ea_hbm, *rest):
        if affine:
            sc_hbm, of_hbm = rest[0], rest[1]
            rest = rest[2:]
        out_hbm, aggr_sh = rest[0], rest[1]
        sc = rest[2:]
        srcb, sc = sc[:R], sc[R:]
        dstb, sc = sc[:R], sc[R:]
        dstS, sc = sc[:R], sc[R:]
        eab, sc = sc[:R2], sc[R2:]
        xrb, sc = sc[:R], sc[R:]
        if TAIL:
            (srct, dstt), sc = sc[:2], sc[2:]
        if affine:
            (scale_v, off_v), sc = sc[:2], sc[2:]
            pltpu.sync_copy(sc_hbm, scale_v)
            pltpu.sync_copy(of_hbm, off_v)
        isem, sc = sc[:R], sc[R:]
        gsem, sc = sc[:R], sc[R:]
        easem, sc = sc[:R2], sc[R2:]
        ssem = sc[:R]
        tsem = sc[R]

        c = lax.axis_index("c")
        s = lax.axis_index("s")
        wid = c * _NS + s
        base = wid * EPW

        @plsc.parallel_loop(0, B)
        def _(i):
            for j in range(D // L):
                xrb[0][i, pl.ds(j * L, L)] = jnp.zeros((L,), jnp.float32)

        NZ = RPT // B
        REM = RPT - NZ * B
        for t in range(NZ):
            pltpu.async_copy(xrb[0], aggr_sh.at[pl.ds(s * RPT + t * B, B)],
                             tsem)
        if REM:
            pltpu.async_copy(xrb[0].at[pl.ds(0, REM)],
                             aggr_sh.at[pl.ds(s * RPT + NZ * B, REM)], tsem)
        for t in range(NZ):
            pltpu.make_async_copy(xrb[0], aggr_sh.at[pl.ds(0, B)],
                                  tsem).wait()
        if REM:
            pltpu.make_async_copy(xrb[0].at[pl.ds(0, REM)],
                                  aggr_sh.at[pl.ds(0, REM)], tsem).wait()
        plsc.subcore_barrier()

        def issue_idx(g, b):
            off = base + g * B
            pltpu.async_copy(src_hbm.at[pl.ds(off, B)], srcb[b], isem[b])
            pltpu.async_copy(dst_hbm.at[pl.ds(off, B)], dstb[b], isem[b])

        def wait_idx(b):
            pltpu.make_async_copy(src_hbm.at[pl.ds(0, B)], srcb[b],
                                  isem[b]).wait()
            pltpu.make_async_copy(dst_hbm.at[pl.ds(0, B)], dstb[b],
                                  isem[b]).wait()

        def issue_ge(g, b, be):
            off = base + g * B
            pltpu.async_copy(ea_hbm.at[pl.ds(off, B)], eab[be], easem[be])
            pltpu.async_copy(x_hbm.at[srcb[b]], xrb[b], gsem[b])

        def wait_ge(b, be):
            pltpu.make_async_copy(ea_hbm.at[pl.ds(0, B)], eab[be],
                                  easem[be]).wait()
            pltpu.make_async_copy(x_hbm.at[srcb[b]], xrb[b], gsem[b]).wait()

        def issue_scatter(b):
            pltpu.make_async_copy(xrb[b], aggr_sh.at[dstS[b]],
                                  ssem[b]).start(add=True)

        def wait_scatter(b):
            pltpu.make_async_copy(xrb[b], aggr_sh.at[dstS[b]],
                                  ssem[b]).wait()

        def save_dst(b):
            for j in range(B // L):
                sl = pl.ds(j * L, L)
                dstS[b][sl] = dstb[b][sl]

        def compute(xr, ea, n):
            if affine:
                sjs = [scale_v[pl.ds(j * L, L)] for j in range(D // L)]
                ojs = [off_v[pl.ds(j * L, L)] for j in range(D // L)]

                @plsc.parallel_loop(0, n, unroll=2)
                def _(i):
                    for j in range(D // L):
                        sl = pl.ds(j * L, L)
                        y = jnp.maximum(xr[i, sl] * sjs[j] + ojs[j], 0.0)
                        xr[i, sl] = jnp.maximum(y + ea[i, sl], 0.0)
            else:
                @plsc.parallel_loop(0, n, unroll=4)
                def _(i):
                    for j in range(D // L):
                        sl = pl.ds(j * L, L)
                        xr[i, sl] = jnp.maximum(xr[i, sl] + ea[i, sl], 0.0)

        def chunk_body(g, b, be, first, last):
            b1 = (b + 1) % R
            be1 = (be + 1) % R2
            if not last:
                wait_idx(b1)
                if not first:
                    wait_scatter(b1)
                issue_ge(g + 1, b1, be1)
                g2 = min(g + 2, FULL - 1) if isinstance(g, int) \
                    else jnp.minimum(g + 2, FULL - 1)
                issue_idx(g2, (b + 2) % R)
            save_dst(b)
            wait_ge(b, be)
            compute(xrb[b], eab[be], B)
            issue_scatter(b)

        issue_idx(0, 0)
        issue_idx(1, 1)
        wait_idx(0)
        issue_ge(0, 0, 0)
        chunk_body(0, 0, 0, True, False)
        chunk_body(1, 1, 1, True, False)

        def group(p, carry):
            g0 = 2 + p * 6
            for t in range(6):
                chunk_body(g0 + t, (2 + t) % R, t % R2, False, False)
            return carry
        lax.fori_loop(0, NI // 6, group, 0)

        for g in range(2 + NI, FULL):
            chunk_body(g, g % R, g % R2, False, g == FULL - 1)
        wait_idx(FULL % R)

        if TAIL:
            wait_scatter(0)
            off = base + FULL * B
            pltpu.sync_copy(src_hbm.at[pl.ds(off, TAIL)], srct)
            pltpu.sync_copy(dst_hbm.at[pl.ds(off, TAIL)], dstt)
            pltpu.sync_copy(ea_hbm.at[pl.ds(off, TAIL)],
                            eab[0].at[pl.ds(0, TAIL)])
            pltpu.async_copy(x_hbm.at[srct], xrb[0].at[pl.ds(0, TAIL)],
                             tsem).wait()
            compute(xrb[0], eab[0], TAIL)
            pltpu.sync_copy(xrb[0].at[pl.ds(0, TAIL)], aggr_sh.at[dstt],
                            add=True)

        for b in range(R):
            if TAIL and b == 0:
                continue
            wait_scatter(b)
        plsc.subcore_barrier()
        pltpu.sync_copy(aggr_sh.at[pl.ds(s * RPT, RPT)],
                        out_hbm.at[c, pl.ds(s * RPT, RPT)])

    if affine:
        return k(x, src, dst, edge_attr, coef[0], coef[1])
    return k(x, src, dst, edge_attr)


_BR = 400


def _mlp_stats_body(count, nsteps, in_affine, out_coef, eps_ref, x_ref, *refs):
    refs = list(refs)
    coef_in = refs.pop(0) if in_affine else None
    a0_ref, a1_ref, wa_ref, ba_ref, wb_ref, bb_ref = refs[:6]
    refs = refs[6:]
    if out_coef:
        lnw_ref, lnb_ref = refs[:2]
        refs = refs[2:]
    h_ref, s_ref, ss_ref = refs[:3]
    coef_out = refs[3] if out_coef else None

    xin = x_ref[...]
    if in_affine:
        xin = jnp.maximum(xin * coef_in[0:1, :] + coef_in[1:2, :], 0.0)
    t = (1.0 + eps_ref[0]) * xin + a0_ref[...] + a1_ref[...]
    u = jnp.dot(t, wa_ref[...], preferred_element_type=jnp.float32)
    u = jnp.maximum(u + ba_ref[...], 0.0)
    h = jnp.dot(u, wb_ref[...], preferred_element_type=jnp.float32)
    h = h + bb_ref[...]
    h_ref[...] = h
    hp = h.reshape(h.shape[0] // 8, 8, h.shape[1])

    @pl.when(pl.program_id(0) == 0)
    def _():
        s_ref[...] = jnp.zeros_like(s_ref)
        ss_ref[...] = jnp.zeros_like(ss_ref)

    s_ref[...] += jnp.sum(hp, axis=0)
    ss_ref[...] += jnp.sum(hp * hp, axis=0)

    if out_coef:
        @pl.when(pl.program_id(0) == nsteps - 1)
        def _():
            mean = jnp.sum(s_ref[...]) / count
            ex2 = jnp.sum(ss_ref[...]) / count
            inv = lax.rsqrt(ex2 - mean * mean + 1e-5)
            scale = inv * lnw_ref[...]
            off = lnb_ref[...] - mean * scale
            pad = jnp.zeros((6, scale.shape[1]), jnp.float32)
            coef_out[...] = jnp.concatenate([scale, off, pad], axis=0)


def _mlp_stats(x, a0, a1, Wa, ba, Wb, bb, eps, ln_w=None, ln_b=None,
               coef=None):
    N, D = x.shape
    G = N // _BR
    out_coef = ln_w is not None
    in_affine = coef is not None
    row_spec = pl.BlockSpec((_BR, D), lambda i: (i, 0))
    full_spec = pl.BlockSpec((D, D), lambda i: (0, 0))
    vec_spec = pl.BlockSpec((1, D), lambda i: (0, 0))
    acc_spec = pl.BlockSpec((8, D), lambda i: (0, 0))

    args = [eps.reshape(1), x]
    in_specs = [pl.BlockSpec(memory_space=pltpu.SMEM), row_spec]
    if in_affine:
        args.append(coef)
        in_specs.append(acc_spec)
    args += [a0, a1, Wa, ba.reshape(1, D), Wb, bb.reshape(1, D)]
    in_specs += [row_spec, row_spec, full_spec, vec_spec, full_spec, vec_spec]
    if out_coef:
        args += [ln_w.reshape(1, D), ln_b.reshape(1, D)]
        in_specs += [vec_spec, vec_spec]

    out_specs = [row_spec, acc_spec, acc_spec]
    out_shape = [
        jax.ShapeDtypeStruct((N, D), jnp.float32),
        jax.ShapeDtypeStruct((8, D), jnp.float32),
        jax.ShapeDtypeStruct((8, D), jnp.float32),
    ]
    if out_coef:
        out_specs.append(acc_spec)
        out_shape.append(jax.ShapeDtypeStruct((8, D), jnp.float32))

    return pl.pallas_call(
        functools.partial(_mlp_stats_body, float(N * D), G, in_affine,
                          out_coef),
        grid=(G,),
        in_specs=in_specs,
        out_specs=out_specs,
        out_shape=out_shape,
    )(*args)


def _norm_body(count, resid, h_ref, s_ref, ss_ref, w_ref, b_ref, *rest):
    if resid:
        x_ref, y_ref = rest
    else:
        (y_ref,) = rest
    mean = jnp.sum(s_ref[...]) / count
    ex2 = jnp.sum(ss_ref[...]) / count
    inv = lax.rsqrt(ex2 - mean * mean + 1e-5)
    y = (h_ref[...] - mean) * inv * w_ref[...] + b_ref[...]
    if resid:
        y = (y + x_ref[...]) * 0.5
    y_ref[...] = jnp.maximum(y, 0.0)


def _norm_relu(h, s, ss, w, b, x=None):
    N, D = h.shape
    G = N // _BR
    resid = x is not None
    row_spec = pl.BlockSpec((_BR, D), lambda i: (i, 0))
    acc_spec = pl.BlockSpec((8, D), lambda i: (0, 0))
    vec_spec = pl.BlockSpec((1, D), lambda i: (0, 0))
    args = [h, s, ss, w.reshape(1, D), b.reshape(1, D)]
    in_specs = [row_spec, acc_spec, acc_spec, vec_spec, vec_spec]
    if resid:
        args.append(x)
        in_specs.append(row_spec)
    return pl.pallas_call(
        functools.partial(_norm_body, float(N * D), resid),
        grid=(G,),
        in_specs=in_specs,
        out_specs=row_spec,
        out_shape=jax.ShapeDtypeStruct((N, D), jnp.float32),
    )(*args)


def kernel(x, edge_index, edge_attr, W1a, b1a, W1b, b1b, eps1, ln1_w, ln1_b,
           W2a, b2a, W2b, b2b, eps2, ln2_w, ln2_b):
    src = edge_index[0]
    dst = edge_index[1]

    agg = _edge_aggregate(x, src, dst, edge_attr)
    h1, s1, ss1, coef1 = _mlp_stats(x, agg[0], agg[1], W1a, b1a, W1b, b1b,
                                    eps1, ln1_w, ln1_b)

    agg2 = _edge_aggregate(h1, src, dst, edge_attr, (coef1[0], coef1[1]))
    h2, s2, ss2 = _mlp_stats(h1, agg2[0], agg2[1], W2a, b2a, W2b, b2b, eps2,
                             coef=coef1)
    out = _norm_relu(h2, s2, ss2, ln2_w, ln2_b, x)
    return out


# speedup vs baseline: 1.0277x; 1.0002x over previous
import functools

import jax
import jax.numpy as jnp
from jax import lax
from jax.experimental import pallas as pl
from jax.experimental.pallas import tpu as pltpu
from jax.experimental.pallas import tpu_sc as plsc

_NC = 2
_NS = 16
_NW = _NC * _NS


def _edge_aggregate(x, src, dst, edge_attr, coef=None):
    N, D = x.shape
    affine = coef is not None
    E = src.shape[0]
    B = 64
    EPW = E // _NW
    FULL = EPW // B
    TAIL = EPW - FULL * B
    assert EPW * _NW == E and TAIL % 8 == 0 and FULL >= 9
    NI = ((FULL - 2) // 6) * 6
    PEEL = FULL - 2 - NI
    if PEEL == 0:
        NI -= 6
        PEEL = 6
    NP = ((N + _NS * 8 - 1) // (_NS * 8)) * (_NS * 8)
    RPT = NP // _NS
    assert RPT % 8 == 0 and (RPT % B) % 8 == 0
    L = 16
    R = 3
    R2 = 2

    mesh = plsc.VectorSubcoreMesh(core_axis_name="c", subcore_axis_name="s")

    scratch = [pltpu.VMEM_SHARED((NP, D), jnp.float32)]
    scratch += [pltpu.VMEM((B,), jnp.int32) for _ in range(R)]
    scratch += [pltpu.VMEM((B,), jnp.int32) for _ in range(R)]
    scratch += [pltpu.VMEM((B,), jnp.int32) for _ in range(R)]
    scratch += [pltpu.VMEM((B, D), jnp.float32) for _ in range(R2)]
    scratch += [pltpu.VMEM((B, D), jnp.float32) for _ in range(R)]
    if TAIL:
        scratch += [pltpu.VMEM((TAIL,), jnp.int32),
                    pltpu.VMEM((TAIL,), jnp.int32)]
    if affine:
        scratch += [pltpu.VMEM((D,), jnp.float32),
                    pltpu.VMEM((D,), jnp.float32)]
    scratch += [pltpu.SemaphoreType.DMA] * (3 * R + R2 + 1)

    @functools.partial(
        pl.kernel,
        out_type=jax.ShapeDtypeStruct((_NC, NP, D), jnp.float32),
        mesh=mesh,
        scratch_types=scratch,
    )
    def k(x_hbm, src_hbm, dst_hbm, ea_hbm, *rest):
        if affine:
            sc_hbm, of_hbm = rest[0], rest[1]
            rest = rest[2:]
        out_hbm, aggr_sh = rest[0], rest[1]
        sc = rest[2:]
        srcb, sc = sc[:R], sc[R:]
        dstb, sc = sc[:R], sc[R:]
        dstS, sc = sc[:R], sc[R:]
        eab, sc = sc[:R2], sc[R2:]
        xrb, sc = sc[:R], sc[R:]
        if TAIL:
            (srct, dstt), sc = sc[:2], sc[2:]
        if affine:
            (scale_v, off_v), sc = sc[:2], sc[2:]
            pltpu.sync_copy(sc_hbm, scale_v)
            pltpu.sync_copy(of_hbm, off_v)
        isem, sc = sc[:R], sc[R:]
        gsem, sc = sc[:R], sc[R:]
        easem, sc = sc[:R2], sc[R2:]
        ssem = sc[:R]
        tsem = sc[R]

        c = lax.axis_index("c")
        s = lax.axis_index("s")
        wid = c * _NS + s
        base = wid * EPW

        @plsc.parallel_loop(0, B)
        def _(i):
            for j in range(D // L):
                xrb[0][i, pl.ds(j * L, L)] = jnp.zeros((L,), jnp.float32)

        NZ = RPT // B
        REM = RPT - NZ * B
        for t in range(NZ):
            pltpu.async_copy(xrb[0], aggr_sh.at[pl.ds(s * RPT + t * B, B)],
                             tsem)
        if REM:
            pltpu.async_copy(xrb[0].at[pl.ds(0, REM)],
                             aggr_sh.at[pl.ds(s * RPT + NZ * B, REM)], tsem)
        for t in range(NZ):
            pltpu.make_async_copy(xrb[0], aggr_sh.at[pl.ds(0, B)],
                                  tsem).wait()
        if REM:
            pltpu.make_async_copy(xrb[0].at[pl.ds(0, REM)],
                                  aggr_sh.at[pl.ds(0, REM)], tsem).wait()
        plsc.subcore_barrier()

        def issue_idx(g, b):
            off = base + g * B
            pltpu.async_copy(src_hbm.at[pl.ds(off, B)], srcb[b], isem[b])
            pltpu.async_copy(dst_hbm.at[pl.ds(off, B)], dstb[b], isem[b])

        def wait_idx(b):
            pltpu.make_async_copy(src_hbm.at[pl.ds(0, B)], srcb[b],
                                  isem[b]).wait()
            pltpu.make_async_copy(dst_hbm.at[pl.ds(0, B)], dstb[b],
                                  isem[b]).wait()

        def issue_ea(g, be):
            off = base + g * B
            pltpu.async_copy(ea_hbm.at[pl.ds(off, B)], eab[be], easem[be])

        def issue_gather(g, b):
            pltpu.async_copy(x_hbm.at[srcb[b]], xrb[b], gsem[b])

        def wait_ge(b, be):
            pltpu.make_async_copy(ea_hbm.at[pl.ds(0, B)], eab[be],
                                  easem[be]).wait()
            pltpu.make_async_copy(x_hbm.at[srcb[b]], xrb[b], gsem[b]).wait()

        def issue_scatter(b):
            pltpu.make_async_copy(xrb[b], aggr_sh.at[dstS[b]],
                                  ssem[b]).start(add=True)

        def wait_scatter(b):
            pltpu.make_async_copy(xrb[b], aggr_sh.at[dstS[b]],
                                  ssem[b]).wait()

        def save_dst(b):
            for j in range(B // L):
                sl = pl.ds(j * L, L)
                dstS[b][sl] = dstb[b][sl]

        def compute(xr, ea, n):
            if affine:
                sjs = [scale_v[pl.ds(j * L, L)] for j in range(D // L)]
                ojs = [off_v[pl.ds(j * L, L)] for j in range(D // L)]

                @plsc.parallel_loop(0, n, unroll=2)
                def _(i):
                    for j in range(D // L):
                        sl = pl.ds(j * L, L)
                        y = jnp.maximum(xr[i, sl] * sjs[j] + ojs[j], 0.0)
                        xr[i, sl] = jnp.maximum(y + ea[i, sl], 0.0)
            else:
                @plsc.parallel_loop(0, n, unroll=4)
                def _(i):
                    for j in range(D // L):
                        sl = pl.ds(j * L, L)
                        xr[i, sl] = jnp.maximum(xr[i, sl] + ea[i, sl], 0.0)

        def chunk_body(g, b, be, first, last):
            b1 = (b + 1) % R
            be1 = (be + 1) % R2
            if not last:
                wait_idx(b1)
                issue_ea(g + 1, be1)
                g2 = min(g + 2, FULL - 1) if isinstance(g, int) \
                    else jnp.minimum(g + 2, FULL - 1)
                issue_idx(g2, (b + 2) % R)
                if not first:
                    wait_scatter(b1)
                issue_gather(g + 1, b1)
            save_dst(b)
            wait_ge(b, be)
            compute(xrb[b], eab[be], B)
            issue_scatter(b)

        issue_idx(0, 0)
        issue_idx(1, 1)
        wait_idx(0)
        issue_ea(0, 0)
        issue_gather(0, 0)
        chunk_body(0, 0, 0, True, False)
        chunk_body(1, 1, 1, True, False)

        def group(p, carry):
            g0 = 2 + p * 6
            for t in range(6):
                chunk_body(g0 + t, (2 + t) % R, t % R2, False, False)
            return carry
        lax.fori_loop(0, NI // 6, group, 0)

        for g in range(2 + NI, FULL):
            chunk_body(g, g % R, g % R2, False, g == FULL - 1)
        wait_idx(FULL % R)

        if TAIL:
            wait_scatter(0)
            off = base + FULL * B
            pltpu.sync_copy(src_hbm.at[pl.ds(off, TAIL)], srct)
            pltpu.sync_copy(dst_hbm.at[pl.ds(off, TAIL)], dstt)
            pltpu.sync_copy(ea_hbm.at[pl.ds(off, TAIL)],
                            eab[0].at[pl.ds(0, TAIL)])
            pltpu.async_copy(x_hbm.at[srct], xrb[0].at[pl.ds(0, TAIL)],
                             tsem).wait()
            compute(xrb[0], eab[0], TAIL)
            pltpu.sync_copy(xrb[0].at[pl.ds(0, TAIL)], aggr_sh.at[dstt],
                            add=True)

        for b in range(R):
            if TAIL and b == 0:
                continue
            wait_scatter(b)
        plsc.subcore_barrier()
        pltpu.sync_copy(aggr_sh.at[pl.ds(s * RPT, RPT)],
                        out_hbm.at[c, pl.ds(s * RPT, RPT)])

    if affine:
        return k(x, src, dst, edge_attr, coef[0], coef[1])
    return k(x, src, dst, edge_attr)


_BR = 400


def _mlp_stats_body(count, nsteps, in_affine, out_coef, eps_ref, x_ref, *refs):
    refs = list(refs)
    coef_in = refs.pop(0) if in_affine else None
    a0_ref, a1_ref, wa_ref, ba_ref, wb_ref, bb_ref = refs[:6]
    refs = refs[6:]
    if out_coef:
        lnw_ref, lnb_ref = refs[:2]
        refs = refs[2:]
    h_ref, s_ref, ss_ref = refs[:3]
    coef_out = refs[3] if out_coef else None

    xin = x_ref[...]
    if in_affine:
        xin = jnp.maximum(xin * coef_in[0:1, :] + coef_in[1:2, :], 0.0)
    t = (1.0 + eps_ref[0]) * xin + a0_ref[...] + a1_ref[...]
    u = jnp.dot(t, wa_ref[...], preferred_element_type=jnp.float32)
    u = jnp.maximum(u + ba_ref[...], 0.0)
    h = jnp.dot(u, wb_ref[...], preferred_element_type=jnp.float32)
    h = h + bb_ref[...]
    h_ref[...] = h
    hp = h.reshape(h.shape[0] // 8, 8, h.shape[1])

    @pl.when(pl.program_id(0) == 0)
    def _():
        s_ref[...] = jnp.zeros_like(s_ref)
        ss_ref[...] = jnp.zeros_like(ss_ref)

    s_ref[...] += jnp.sum(hp, axis=0)
    ss_ref[...] += jnp.sum(hp * hp, axis=0)

    if out_coef:
        @pl.when(pl.program_id(0) == nsteps - 1)
        def _():
            mean = jnp.sum(s_ref[...]) / count
            ex2 = jnp.sum(ss_ref[...]) / count
            inv = lax.rsqrt(ex2 - mean * mean + 1e-5)
            scale = inv * lnw_ref[...]
            off = lnb_ref[...] - mean * scale
            pad = jnp.zeros((6, scale.shape[1]), jnp.float32)
            coef_out[...] = jnp.concatenate([scale, off, pad], axis=0)


def _mlp_stats(x, a0, a1, Wa, ba, Wb, bb, eps, ln_w=None, ln_b=None,
               coef=None):
    N, D = x.shape
    G = N // _BR
    out_coef = ln_w is not None
    in_affine = coef is not None
    row_spec = pl.BlockSpec((_BR, D), lambda i: (i, 0))
    full_spec = pl.BlockSpec((D, D), lambda i: (0, 0))
    vec_spec = pl.BlockSpec((1, D), lambda i: (0, 0))
    acc_spec = pl.BlockSpec((8, D), lambda i: (0, 0))

    args = [eps.reshape(1), x]
    in_specs = [pl.BlockSpec(memory_space=pltpu.SMEM), row_spec]
    if in_affine:
        args.append(coef)
        in_specs.append(acc_spec)
    args += [a0, a1, Wa, ba.reshape(1, D), Wb, bb.reshape(1, D)]
    in_specs += [row_spec, row_spec, full_spec, vec_spec, full_spec, vec_spec]
    if out_coef:
        args += [ln_w.reshape(1, D), ln_b.reshape(1, D)]
        in_specs += [vec_spec, vec_spec]

    out_specs = [row_spec, acc_spec, acc_spec]
    out_shape = [
        jax.ShapeDtypeStruct((N, D), jnp.float32),
        jax.ShapeDtypeStruct((8, D), jnp.float32),
        jax.ShapeDtypeStruct((8, D), jnp.float32),
    ]
    if out_coef:
        out_specs.append(acc_spec)
        out_shape.append(jax.ShapeDtypeStruct((8, D), jnp.float32))

    return pl.pallas_call(
        functools.partial(_mlp_stats_body, float(N * D), G, in_affine,
                          out_coef),
        grid=(G,),
        in_specs=in_specs,
        out_specs=out_specs,
        out_shape=out_shape,
    )(*args)


def _norm_body(count, resid, h_ref, s_ref, ss_ref, w_ref, b_ref, *rest):
    if resid:
        x_ref, y_ref = rest
    else:
        (y_ref,) = rest
    mean = jnp.sum(s_ref[...]) / count
    ex2 = jnp.sum(ss_ref[...]) / count
    inv = lax.rsqrt(ex2 - mean * mean + 1e-5)
    y = (h_ref[...] - mean) * inv * w_ref[...] + b_ref[...]
    if resid:
        y = (y + x_ref[...]) * 0.5
    y_ref[...] = jnp.maximum(y, 0.0)


def _norm_relu(h, s, ss, w, b, x=None):
    N, D = h.shape
    G = N // _BR
    resid = x is not None
    row_spec = pl.BlockSpec((_BR, D), lambda i: (i, 0))
    acc_spec = pl.BlockSpec((8, D), lambda i: (0, 0))
    vec_spec = pl.BlockSpec((1, D), lambda i: (0, 0))
    args = [h, s, ss, w.reshape(1, D), b.reshape(1, D)]
    in_specs = [row_spec, acc_spec, acc_spec, vec_spec, vec_spec]
    if resid:
        args.append(x)
        in_specs.append(row_spec)
    return pl.pallas_call(
        functools.partial(_norm_body, float(N * D), resid),
        grid=(G,),
        in_specs=in_specs,
        out_specs=row_spec,
        out_shape=jax.ShapeDtypeStruct((N, D), jnp.float32),
    )(*args)


def kernel(x, edge_index, edge_attr, W1a, b1a, W1b, b1b, eps1, ln1_w, ln1_b,
           W2a, b2a, W2b, b2b, eps2, ln2_w, ln2_b):
    src = edge_index[0]
    dst = edge_index[1]

    agg = _edge_aggregate(x, src, dst, edge_attr)
    h1, s1, ss1, coef1 = _mlp_stats(x, agg[0], agg[1], W1a, b1a, W1b, b1b,
                                    eps1, ln1_w, ln1_b)

    agg2 = _edge_aggregate(h1, src, dst, edge_attr, (coef1[0], coef1[1]))
    h2, s2, ss2 = _mlp_stats(h1, agg2[0], agg2[1], W2a, b2a, W2b, b2b, eps2,
                             coef=coef1)
    out = _norm_relu(h2, s2, ss2, ln2_w, ln2_b, x)
    return out


# speedup vs baseline: 1.0359x; 1.0080x over previous
import functools

import jax
import jax.numpy as jnp
from jax import lax
from jax.experimental import pallas as pl
from jax.experimental.pallas import tpu as pltpu
from jax.experimental.pallas import tpu_sc as plsc

_NC = 2
_NS = 16
_NW = _NC * _NS


def _edge_aggregate(x, src, dst, edge_attr, coef=None):
    N, D = x.shape
    affine = coef is not None
    E = src.shape[0]
    B = 64
    EPW = E // _NW
    FULL = EPW // B
    TAIL = EPW - FULL * B
    assert EPW * _NW == E and TAIL % 8 == 0 and FULL >= 9
    NI = ((FULL - 2) // 6) * 6
    PEEL = FULL - 2 - NI
    if PEEL == 0:
        NI -= 6
        PEEL = 6
    NP = ((N + _NS * 8 - 1) // (_NS * 8)) * (_NS * 8)
    RPT = NP // _NS
    assert RPT % 8 == 0 and (RPT % B) % 8 == 0
    L = 16
    R = 3
    R2 = 2

    mesh = plsc.VectorSubcoreMesh(core_axis_name="c", subcore_axis_name="s")

    scratch = [pltpu.VMEM_SHARED((NP, D), jnp.float32)]
    scratch += [pltpu.VMEM((B,), jnp.int32) for _ in range(R)]
    scratch += [pltpu.VMEM((B,), jnp.int32) for _ in range(R)]
    scratch += [pltpu.VMEM((B,), jnp.int32) for _ in range(R)]
    scratch += [pltpu.VMEM((B, D), jnp.float32) for _ in range(R2)]
    scratch += [pltpu.VMEM((B, D), jnp.float32) for _ in range(R)]
    if TAIL:
        scratch += [pltpu.VMEM((TAIL,), jnp.int32),
                    pltpu.VMEM((TAIL,), jnp.int32)]
    if affine:
        scratch += [pltpu.VMEM((D,), jnp.float32),
                    pltpu.VMEM((D,), jnp.float32)]
    scratch += [pltpu.SemaphoreType.DMA] * (3 * R + R2 + 1)

    @functools.partial(
        pl.kernel,
        out_type=jax.ShapeDtypeStruct((_NC, NP, D), jnp.float32),
        mesh=mesh,
        scratch_types=scratch,
    )
    def k(x_hbm, src_hbm, dst_hbm, ea_hbm, *rest):
        if affine:
            sc_hbm, of_hbm = rest[0], rest[1]
            rest = rest[2:]
        out_hbm, aggr_sh = rest[0], rest[1]
        sc = rest[2:]
        srcb, sc = sc[:R], sc[R:]
        dstb, sc = sc[:R], sc[R:]
        dstS, sc = sc[:R], sc[R:]
        eab, sc = sc[:R2], sc[R2:]
        xrb, sc = sc[:R], sc[R:]
        if TAIL:
            (srct, dstt), sc = sc[:2], sc[2:]
        if affine:
            (scale_v, off_v), sc = sc[:2], sc[2:]
            pltpu.sync_copy(sc_hbm, scale_v)
            pltpu.sync_copy(of_hbm, off_v)
        isem, sc = sc[:R], sc[R:]
        gsem, sc = sc[:R], sc[R:]
        easem, sc = sc[:R2], sc[R2:]
        ssem = sc[:R]
        tsem = sc[R]

        c = lax.axis_index("c")
        s = lax.axis_index("s")
        wid = c * _NS + s
        base = wid * EPW

        @plsc.parallel_loop(0, B)
        def _(i):
            for j in range(D // L):
                xrb[0][i, pl.ds(j * L, L)] = jnp.zeros((L,), jnp.float32)

        NZ = RPT // B
        REM = RPT - NZ * B
        for t in range(NZ):
            pltpu.async_copy(xrb[0], aggr_sh.at[pl.ds(s * RPT + t * B, B)],
                             tsem)
        if REM:
            pltpu.async_copy(xrb[0].at[pl.ds(0, REM)],
                             aggr_sh.at[pl.ds(s * RPT + NZ * B, REM)], tsem)
        for t in range(NZ):
            pltpu.make_async_copy(xrb[0], aggr_sh.at[pl.ds(0, B)],
                                  tsem).wait()
        if REM:
            pltpu.make_async_copy(xrb[0].at[pl.ds(0, REM)],
                                  aggr_sh.at[pl.ds(0, REM)], tsem).wait()
        plsc.subcore_barrier()

        def issue_idx(g, b):
            off = base + g * B
            pltpu.async_copy(src_hbm.at[pl.ds(off, B)], srcb[b], isem[b])
            pltpu.async_copy(dst_hbm.at[pl.ds(off, B)], dstb[b], isem[b])

        def wait_idx(b):
            pltpu.make_async_copy(src_hbm.at[pl.ds(0, B)], srcb[b],
                                  isem[b]).wait()
            pltpu.make_async_copy(dst_hbm.at[pl.ds(0, B)], dstb[b],
                                  isem[b]).wait()

        def issue_ea(g, be):
            off = base + g * B
            pltpu.async_copy(ea_hbm.at[pl.ds(off, B)], eab[be], easem[be])

        def issue_gather(g, b):
            pltpu.async_copy(x_hbm.at[srcb[b]], xrb[b], gsem[b])

        def wait_ge(b, be):
            pltpu.make_async_copy(ea_hbm.at[pl.ds(0, B)], eab[be],
                                  easem[be]).wait()
            pltpu.make_async_copy(x_hbm.at[srcb[b]], xrb[b], gsem[b]).wait()

        def issue_scatter(b):
            pltpu.make_async_copy(xrb[b], aggr_sh.at[dstS[b]],
                                  ssem[b]).start(add=True)

        def wait_scatter(b):
            pltpu.make_async_copy(xrb[b], aggr_sh.at[dstS[b]],
                                  ssem[b]).wait()

        def save_dst(b):
            for j in range(B // L):
                sl = pl.ds(j * L, L)
                dstS[b][sl] = dstb[b][sl]

        def compute(xr, ea, n):
            if affine:
                sjs = [scale_v[pl.ds(j * L, L)] for j in range(D // L)]
                ojs = [off_v[pl.ds(j * L, L)] for j in range(D // L)]

                @plsc.parallel_loop(0, n, unroll=4)
                def _(i):
                    for j in range(D // L):
                        sl = pl.ds(j * L, L)
                        y = jnp.maximum(xr[i, sl] * sjs[j] + ojs[j], 0.0)
                        xr[i, sl] = jnp.maximum(y + ea[i, sl], 0.0)
            else:
                @plsc.parallel_loop(0, n, unroll=4)
                def _(i):
                    for j in range(D // L):
                        sl = pl.ds(j * L, L)
                        xr[i, sl] = jnp.maximum(xr[i, sl] + ea[i, sl], 0.0)

        def chunk_body(g, b, be, first, last):
            b1 = (b + 1) % R
            be1 = (be + 1) % R2
            if not last:
                wait_idx(b1)
                issue_ea(g + 1, be1)
                g2 = min(g + 2, FULL - 1) if isinstance(g, int) \
                    else jnp.minimum(g + 2, FULL - 1)
                issue_idx(g2, (b + 2) % R)
                if not first:
                    wait_scatter(b1)
                issue_gather(g + 1, b1)
            save_dst(b)
            wait_ge(b, be)
            compute(xrb[b], eab[be], B)
            issue_scatter(b)

        issue_idx(0, 0)
        issue_idx(1, 1)
        wait_idx(0)
        issue_ea(0, 0)
        issue_gather(0, 0)
        chunk_body(0, 0, 0, True, False)
        chunk_body(1, 1, 1, True, False)

        def group(p, carry):
            g0 = 2 + p * 6
            for t in range(6):
                chunk_body(g0 + t, (2 + t) % R, t % R2, False, False)
            return carry
        lax.fori_loop(0, NI // 6, group, 0)

        for g in range(2 + NI, FULL):
            chunk_body(g, g % R, g % R2, False, g == FULL - 1)
        wait_idx(FULL % R)

        if TAIL:
            wait_scatter(0)
            off = base + FULL * B
            pltpu.sync_copy(src_hbm.at[pl.ds(off, TAIL)], srct)
            pltpu.sync_copy(dst_hbm.at[pl.ds(off, TAIL)], dstt)
            pltpu.sync_copy(ea_hbm.at[pl.ds(off, TAIL)],
                            eab[0].at[pl.ds(0, TAIL)])
            pltpu.async_copy(x_hbm.at[srct], xrb[0].at[pl.ds(0, TAIL)],
                             tsem).wait()
            compute(xrb[0], eab[0], TAIL)
            pltpu.sync_copy(xrb[0].at[pl.ds(0, TAIL)], aggr_sh.at[dstt],
                            add=True)

        for b in range(R):
            if TAIL and b == 0:
                continue
            wait_scatter(b)
        plsc.subcore_barrier()
        pltpu.sync_copy(aggr_sh.at[pl.ds(s * RPT, RPT)],
                        out_hbm.at[c, pl.ds(s * RPT, RPT)])

    if affine:
        return k(x, src, dst, edge_attr, coef[0], coef[1])
    return k(x, src, dst, edge_attr)


_BR = 400


def _mlp_stats_body(count, nsteps, in_affine, out_coef, eps_ref, x_ref, *refs):
    refs = list(refs)
    coef_in = refs.pop(0) if in_affine else None
    a0_ref, a1_ref, wa_ref, ba_ref, wb_ref, bb_ref = refs[:6]
    refs = refs[6:]
    if out_coef:
        lnw_ref, lnb_ref = refs[:2]
        refs = refs[2:]
    h_ref, s_ref, ss_ref = refs[:3]
    coef_out = refs[3] if out_coef else None

    xin = x_ref[...]
    if in_affine:
        xin = jnp.maximum(xin * coef_in[0:1, :] + coef_in[1:2, :], 0.0)
    t = (1.0 + eps_ref[0]) * xin + a0_ref[...] + a1_ref[...]
    u = jnp.dot(t, wa_ref[...], preferred_element_type=jnp.float32)
    u = jnp.maximum(u + ba_ref[...], 0.0)
    h = jnp.dot(u, wb_ref[...], preferred_element_type=jnp.float32)
    h = h + bb_ref[...]
    h_ref[...] = h
    hp = h.reshape(h.shape[0] // 8, 8, h.shape[1])

    @pl.when(pl.program_id(0) == 0)
    def _():
        s_ref[...] = jnp.zeros_like(s_ref)
        ss_ref[...] = jnp.zeros_like(ss_ref)

    s_ref[...] += jnp.sum(hp, axis=0)
    ss_ref[...] += jnp.sum(hp * hp, axis=0)

    if out_coef:
        @pl.when(pl.program_id(0) == nsteps - 1)
        def _():
            mean = jnp.sum(s_ref[...]) / count
            ex2 = jnp.sum(ss_ref[...]) / count
            inv = lax.rsqrt(ex2 - mean * mean + 1e-5)
            scale = inv * lnw_ref[...]
            off = lnb_ref[...] - mean * scale
            pad = jnp.zeros((6, scale.shape[1]), jnp.float32)
            coef_out[...] = jnp.concatenate([scale, off, pad], axis=0)


def _mlp_stats(x, a0, a1, Wa, ba, Wb, bb, eps, ln_w=None, ln_b=None,
               coef=None):
    N, D = x.shape
    G = N // _BR
    out_coef = ln_w is not None
    in_affine = coef is not None
    row_spec = pl.BlockSpec((_BR, D), lambda i: (i, 0))
    full_spec = pl.BlockSpec((D, D), lambda i: (0, 0))
    vec_spec = pl.BlockSpec((1, D), lambda i: (0, 0))
    acc_spec = pl.BlockSpec((8, D), lambda i: (0, 0))

    args = [eps.reshape(1), x]
    in_specs = [pl.BlockSpec(memory_space=pltpu.SMEM), row_spec]
    if in_affine:
        args.append(coef)
        in_specs.append(acc_spec)
    args += [a0, a1, Wa, ba.reshape(1, D), Wb, bb.reshape(1, D)]
    in_specs += [row_spec, row_spec, full_spec, vec_spec, full_spec, vec_spec]
    if out_coef:
        args += [ln_w.reshape(1, D), ln_b.reshape(1, D)]
        in_specs += [vec_spec, vec_spec]

    out_specs = [row_spec, acc_spec, acc_spec]
    out_shape = [
        jax.ShapeDtypeStruct((N, D), jnp.float32),
        jax.ShapeDtypeStruct((8, D), jnp.float32),
        jax.ShapeDtypeStruct((8, D), jnp.float32),
    ]
    if out_coef:
        out_specs.append(acc_spec)
        out_shape.append(jax.ShapeDtypeStruct((8, D), jnp.float32))

    return pl.pallas_call(
        functools.partial(_mlp_stats_body, float(N * D), G, in_affine,
                          out_coef),
        grid=(G,),
        in_specs=in_specs,
        out_specs=out_specs,
        out_shape=out_shape,
    )(*args)


def _norm_body(count, resid, h_ref, s_ref, ss_ref, w_ref, b_ref, *rest):
    if resid:
        x_ref, y_ref = rest
    else:
        (y_ref,) = rest
    mean = jnp.sum(s_ref[...]) / count
    ex2 = jnp.sum(ss_ref[...]) / count
    inv = lax.rsqrt(ex2 - mean * mean + 1e-5)
    y = (h_ref[...] - mean) * inv * w_ref[...] + b_ref[...]
    if resid:
        y = (y + x_ref[...]) * 0.5
    y_ref[...] = jnp.maximum(y, 0.0)


def _norm_relu(h, s, ss, w, b, x=None):
    N, D = h.shape
    G = N // _BR
    resid = x is not None
    row_spec = pl.BlockSpec((_BR, D), lambda i: (i, 0))
    acc_spec = pl.BlockSpec((8, D), lambda i: (0, 0))
    vec_spec = pl.BlockSpec((1, D), lambda i: (0, 0))
    args = [h, s, ss, w.reshape(1, D), b.reshape(1, D)]
    in_specs = [row_spec, acc_spec, acc_spec, vec_spec, vec_spec]
    if resid:
        args.append(x)
        in_specs.append(row_spec)
    return pl.pallas_call(
        functools.partial(_norm_body, float(N * D), resid),
        grid=(G,),
        in_specs=in_specs,
        out_specs=row_spec,
        out_shape=jax.ShapeDtypeStruct((N, D), jnp.float32),
    )(*args)


def kernel(x, edge_index, edge_attr, W1a, b1a, W1b, b1b, eps1, ln1_w, ln1_b,
           W2a, b2a, W2b, b2b, eps2, ln2_w, ln2_b):
    src = edge_index[0]
    dst = edge_index[1]

    agg = _edge_aggregate(x, src, dst, edge_attr)
    h1, s1, ss1, coef1 = _mlp_stats(x, agg[0], agg[1], W1a, b1a, W1b, b1b,
                                    eps1, ln1_w, ln1_b)

    agg2 = _edge_aggregate(h1, src, dst, edge_attr, (coef1[0], coef1[1]))
    h2, s2, ss2 = _mlp_stats(h1, agg2[0], agg2[1], W2a, b2a, W2b, b2b, eps2,
                             coef=coef1)
    out = _norm_relu(h2, s2, ss2, ln2_w, ln2_b, x)
    return out
